# Initial kernel scaffold; baseline (speedup 1.0000x reference)
import jax
import jax.numpy as jnp
from jax.experimental import pallas as pl


def kernel(x, edge_index, batch, W1, b1, W2, b2, W3, b3, lin_W, lin_b):
    raise NotImplementedError("write your pallas kernel here")


# speedup vs baseline: 31.8872x; 31.8872x over previous
import functools

import jax
import jax.numpy as jnp
from jax import lax
from jax.experimental import pallas as pl
from jax.experimental.pallas import tpu as pltpu
from jax.experimental.pallas import tpu_sc as plsc

N_NODES = 10000
X_DIM = 128
HID = 64
N_GRAPHS = 32

NC = 2
NS = 16
NW = NC * NS
C = 128

NCH_E = 80
E_PAD = NW * NCH_E * C
ROWS_ACC = N_NODES + 112
RPT_ACC = ROWS_ACC // NS

NCH_B = 3
B_PAD = NW * NCH_B * C
NCH_H = NCH_E + NCH_B
CNT_GROUPS = NS
CNT_ROWS = N_GRAPHS + 16
ROWS_HIST = ROWS_ACC + CNT_GROUPS * CNT_ROWS
RPT_HIST = ROWS_HIST // NS

NCH_P = 4
P_PAD = NW * NCH_P * C
ROWS_POOL = CNT_GROUPS * CNT_ROWS
RPT_POOL = ROWS_POOL // NS

_MESH = plsc.VectorSubcoreMesh(core_axis_name="c", subcore_axis_name="s")
_SC_PARAMS = pltpu.CompilerParams(use_tc_tiling_on_sc=False)


@functools.partial(
    pl.kernel,
    out_type=jax.ShapeDtypeStruct((NC * ROWS_HIST, 16), jnp.float32),
    mesh=_MESH,
    compiler_params=_SC_PARAMS,
    scratch_types=[
        pltpu.VMEM((NCH_H, C), jnp.int32),
        pltpu.VMEM((C, 16), jnp.float32),
        pltpu.VMEM_SHARED((ROWS_HIST, 16), jnp.float32),
        pltpu.SemaphoreType.DMA,
    ],
)
def _hist_pass(idx_hbm, ones_hbm, zeros_hbm, out_hbm, idx_v, ones_v, acc, sem):
    cid = lax.axis_index("c")
    sid = lax.axis_index("s")
    wid = cid * NS + sid
    pltpu.sync_copy(idx_hbm.at[wid], idx_v)
    pltpu.sync_copy(ones_hbm, ones_v)
    pltpu.sync_copy(zeros_hbm.at[pl.ds(sid * RPT_HIST, RPT_HIST)],
                    acc.at[pl.ds(sid * RPT_HIST, RPT_HIST)])
    plsc.subcore_barrier()
    pending = []
    for j in range(NCH_H):
        pending.append(
            pltpu.async_copy(ones_v, acc.at[idx_v.at[j]], sem, add=True))
        if len(pending) >= 16:
            for cp in pending:
                cp.wait()
            pending = []
    for cp in pending:
        cp.wait()
    plsc.subcore_barrier()
    pltpu.sync_copy(acc.at[pl.ds(sid * RPT_HIST, RPT_HIST)],
                    out_hbm.at[pl.ds(cid * ROWS_HIST + sid * RPT_HIST, RPT_HIST)])


def _make_scatter_pass(nch, n_src_rows, acc_rows):
    rpt = acc_rows // NS

    @functools.partial(
        pl.kernel,
        out_type=jax.ShapeDtypeStruct((NC * acc_rows, HID), jnp.float32),
        mesh=_MESH,
        compiler_params=_SC_PARAMS,
        scratch_types=[
            pltpu.VMEM((nch, C), jnp.int32),
            pltpu.VMEM((nch, C), jnp.int32),
            pltpu.VMEM((C, HID), jnp.float32),
            pltpu.VMEM((C, HID), jnp.float32),
            pltpu.VMEM_SHARED((acc_rows, HID), jnp.float32),
            pltpu.SemaphoreType.DMA,
            pltpu.SemaphoreType.DMA,
            pltpu.SemaphoreType.DMA,
            pltpu.SemaphoreType.DMA,
        ],
    )
    def _pass(src_hbm, dst_hbm, y_hbm, zeros_hbm, out_hbm,
              src_v, dst_v, rows0, rows1, acc, g0, g1, s0, s1):
        cid = lax.axis_index("c")
        sid = lax.axis_index("s")
        wid = cid * NS + sid
        pltpu.sync_copy(src_hbm.at[wid], src_v)
        pltpu.sync_copy(dst_hbm.at[wid], dst_v)
        pltpu.sync_copy(zeros_hbm.at[pl.ds(sid * rpt, rpt)],
                        acc.at[pl.ds(sid * rpt, rpt)])
        plsc.subcore_barrier()

        rows = (rows0, rows1)
        gsem = (g0, g1)
        ssem = (s0, s1)
        pltpu.async_copy(y_hbm.at[src_v.at[0]], rows0, g0)
        pltpu.async_copy(y_hbm.at[src_v.at[1]], rows1, g1)

        def step(i, carry):
            for b in range(2):
                j = 2 * i + b
                pltpu.make_async_copy(y_hbm.at[src_v.at[j]], rows[b],
                                      gsem[b]).wait()
                cp = pltpu.async_copy(rows[b], acc.at[dst_v.at[j]], ssem[b],
                                      add=True)
                cp.wait()

                @pl.when(j + 2 < nch)
                def _():
                    pltpu.async_copy(y_hbm.at[src_v.at[j + 2]], rows[b],
                                     gsem[b])
            return carry

        lax.fori_loop(0, nch // 2, step, 0)
        plsc.subcore_barrier()
        pltpu.sync_copy(acc.at[pl.ds(sid * rpt, rpt)],
                        out_hbm.at[pl.ds(cid * acc_rows + sid * rpt, rpt)])

    del n_src_rows
    return _pass


_edge_pass = _make_scatter_pass(NCH_E, N_NODES, ROWS_ACC)
_pool_pass = _make_scatter_pass(NCH_P, N_NODES, ROWS_POOL)


_BR = 1000
_GRID = N_NODES // _BR


def _mm1_body(x_ref, w_ref, o_ref):
    o_ref[...] = jnp.dot(x_ref[...], w_ref[...],
                         preferred_element_type=jnp.float32)


def _mm1(x, w1):
    return pl.pallas_call(
        _mm1_body,
        grid=(_GRID,),
        in_specs=[
            pl.BlockSpec((_BR, X_DIM), lambda i: (i, 0)),
            pl.BlockSpec((X_DIM, HID), lambda i: (0, 0)),
        ],
        out_specs=pl.BlockSpec((_BR, HID), lambda i: (i, 0)),
        out_shape=jax.ShapeDtypeStruct((N_NODES, HID), jnp.float32),
    )(x, w1)


def _scales_body(h_ref, dinv_ref, icnt_ref):
    hp = h_ref[...]
    t = hp[0] + hp[1]
    deg = t[:ROWS_ACC] + 1.0
    dinv_ref[...] = lax.rsqrt(deg)
    cnt = jnp.sum(t[ROWS_ACC:].reshape(CNT_GROUPS, CNT_ROWS, 16), axis=0)
    icnt_ref[...] = 1.0 / jnp.maximum(cnt, 1.0)


def _scales(hist_parts):
    return pl.pallas_call(
        _scales_body,
        out_shape=(
            jax.ShapeDtypeStruct((ROWS_ACC, 16), jnp.float32),
            jax.ShapeDtypeStruct((CNT_ROWS, 16), jnp.float32),
        ),
    )(hist_parts)


def _scale_y_body(xw_ref, dinv_ref, y_ref):
    y_ref[...] = xw_ref[...] * dinv_ref[:, 0:1]


def _scale_y(xw, dinv_tab):
    return pl.pallas_call(
        _scale_y_body,
        grid=(_GRID,),
        in_specs=[
            pl.BlockSpec((_BR, HID), lambda i: (i, 0)),
            pl.BlockSpec((_BR, 16), lambda i: (i, 0)),
        ],
        out_specs=pl.BlockSpec((_BR, HID), lambda i: (i, 0)),
        out_shape=jax.ShapeDtypeStruct((N_NODES, HID), jnp.float32),
    )(xw, dinv_tab)


def _combine_mm_body(p_ref, xw_ref, dinv_ref, b_ref, w_ref, xwn_ref, yn_ref):
    dv = dinv_ref[:, 0:1]
    raw = p_ref[0] + p_ref[1]
    h = raw * dv + xw_ref[...] * (dv * dv) + b_ref[...]
    h = jnp.maximum(h, 0.0)
    xwn = jnp.dot(h, w_ref[...], preferred_element_type=jnp.float32)
    xwn_ref[...] = xwn
    yn_ref[...] = xwn * dv


def _combine_mm(parts, xw, dinv_tab, b, w_next):
    return pl.pallas_call(
        _combine_mm_body,
        grid=(_GRID,),
        in_specs=[
            pl.BlockSpec((NC, _BR, HID), lambda i: (0, i, 0)),
            pl.BlockSpec((_BR, HID), lambda i: (i, 0)),
            pl.BlockSpec((_BR, 16), lambda i: (i, 0)),
            pl.BlockSpec((1, HID), lambda i: (0, 0)),
            pl.BlockSpec((HID, HID), lambda i: (0, 0)),
        ],
        out_specs=(
            pl.BlockSpec((_BR, HID), lambda i: (i, 0)),
            pl.BlockSpec((_BR, HID), lambda i: (i, 0)),
        ),
        out_shape=(
            jax.ShapeDtypeStruct((N_NODES, HID), jnp.float32),
            jax.ShapeDtypeStruct((N_NODES, HID), jnp.float32),
        ),
    )(parts, xw, dinv_tab, b, w_next)


def _combine_last_body(p_ref, xw_ref, dinv_ref, b_ref, h_ref):
    dv = dinv_ref[:, 0:1]
    raw = p_ref[0] + p_ref[1]
    h_ref[...] = raw * dv + xw_ref[...] * (dv * dv) + b_ref[...]


def _combine_last(parts, xw, dinv_tab, b):
    return pl.pallas_call(
        _combine_last_body,
        grid=(_GRID,),
        in_specs=[
            pl.BlockSpec((NC, _BR, HID), lambda i: (0, i, 0)),
            pl.BlockSpec((_BR, HID), lambda i: (i, 0)),
            pl.BlockSpec((_BR, 16), lambda i: (i, 0)),
            pl.BlockSpec((1, HID), lambda i: (0, 0)),
        ],
        out_specs=pl.BlockSpec((_BR, HID), lambda i: (i, 0)),
        out_shape=jax.ShapeDtypeStruct((N_NODES, HID), jnp.float32),
    )(parts, xw, dinv_tab, b)


def _final_body(pool_ref, icnt_ref, lw_ref, lb_ref, o_ref):
    s = jnp.sum(pool_ref[...].reshape(NC * CNT_GROUPS, CNT_ROWS, HID), axis=0)
    g = s[:N_GRAPHS] * icnt_ref[:N_GRAPHS, 0:1]
    z = jnp.dot(g, lw_ref[...], preferred_element_type=jnp.float32)
    z = z + lb_ref[...]
    o_ref[...] = jax.nn.sigmoid(z)


def _final(pool_parts, icnt, lin_w, lin_b):
    return pl.pallas_call(
        _final_body,
        out_shape=jax.ShapeDtypeStruct((N_GRAPHS, 1), jnp.float32),
    )(pool_parts, icnt, lin_w, lin_b)


def kernel(x, edge_index, batch, W1, b1, W2, b2, W3, b3, lin_W, lin_b):
    n_edges = edge_index.shape[1]
    src = edge_index[0].astype(jnp.int32)
    dst = edge_index[1].astype(jnp.int32)
    batch = batch.astype(jnp.int32)

    pe = E_PAD - n_edges
    are = jnp.arange(pe, dtype=jnp.int32)
    src_p = jnp.concatenate([src, are % N_NODES]).reshape(NW, NCH_E, C)
    dst_p = jnp.concatenate([dst, N_NODES + (are % 16)]).reshape(NW, NCH_E, C)

    pb = B_PAD - N_NODES
    arb = jnp.arange(pb, dtype=jnp.int32)
    wid_of = (jnp.arange(B_PAD, dtype=jnp.int32) // (NCH_B * C)) % NS
    bat_flat = jnp.concatenate(
        [batch, N_GRAPHS + (arb % 16)])
    bat_idx = (ROWS_ACC + wid_of * CNT_ROWS + bat_flat).reshape(NW, NCH_B, C)
    hist_idx = jnp.concatenate([dst_p, bat_idx], axis=1)

    pp = P_PAD - N_NODES
    arp = jnp.arange(pp, dtype=jnp.int32)
    pwid_of = (jnp.arange(P_PAD, dtype=jnp.int32) // (NCH_P * C)) % NS
    pool_src = jnp.concatenate(
        [jnp.arange(N_NODES, dtype=jnp.int32), arp % N_NODES]
    ).reshape(NW, NCH_P, C)
    pool_bin = jnp.concatenate([batch, N_GRAPHS + (arp % 16)])
    pool_dst = (pwid_of * CNT_ROWS + pool_bin).reshape(NW, NCH_P, C)

    ones16 = jnp.ones((C, 16), jnp.float32)
    z_hist = jnp.zeros((ROWS_HIST, 16), jnp.float32)
    z_acc = jnp.zeros((ROWS_ACC, HID), jnp.float32)
    z_pool = jnp.zeros((ROWS_POOL, HID), jnp.float32)
    b1r = b1.reshape(1, HID)
    b2r = b2.reshape(1, HID)
    b3r = b3.reshape(1, HID)

    hist = _hist_pass(hist_idx, ones16, z_hist)
    xw1 = _mm1(x, W1)
    hist3 = hist.reshape(NC, ROWS_HIST, 16)
    dinv_tab, icnt = _scales(hist3)
    y1 = _scale_y(xw1, dinv_tab)

    p1 = _edge_pass(src_p, dst_p, y1, z_acc)
    xw2, y2 = _combine_mm(p1.reshape(NC, ROWS_ACC, HID), xw1, dinv_tab,
                          b1r, W2)
    p2 = _edge_pass(src_p, dst_p, y2, z_acc)
    xw3, y3 = _combine_mm(p2.reshape(NC, ROWS_ACC, HID), xw2, dinv_tab,
                          b2r, W3)
    p3 = _edge_pass(src_p, dst_p, y3, z_acc)
    h3 = _combine_last(p3.reshape(NC, ROWS_ACC, HID), xw3, dinv_tab, b3r)

    pp_ = _pool_pass(pool_src, pool_dst, h3, z_pool)
    out = _final(pp_.reshape(NC * ROWS_POOL, HID), icnt,
                 lin_W, lin_b.reshape(1, 1))
    return out


# speedup vs baseline: 34.7446x; 1.0896x over previous
import functools

import jax
import jax.numpy as jnp
from jax import lax
from jax.experimental import pallas as pl
from jax.experimental.pallas import tpu as pltpu
from jax.experimental.pallas import tpu_sc as plsc

N_NODES = 10000
X_DIM = 128
HID = 64
N_GRAPHS = 32

NC = 2
NS = 16
NW = NC * NS
C = 128

NCH_E = 80
E_PAD = NW * NCH_E * C
ROWS_ACC = N_NODES + 112
RPT_ACC = ROWS_ACC // NS
NBUF = 4

_MESH = plsc.VectorSubcoreMesh(core_axis_name="c", subcore_axis_name="s")
_SC_PARAMS = pltpu.CompilerParams(use_tc_tiling_on_sc=False)


@functools.partial(
    pl.kernel,
    out_type=jax.ShapeDtypeStruct((NC * ROWS_ACC, 16), jnp.float32),
    mesh=_MESH,
    compiler_params=_SC_PARAMS,
    scratch_types=[
        pltpu.VMEM((NCH_E, C), jnp.int32),
        pltpu.VMEM((C, 16), jnp.float32),
        pltpu.VMEM_SHARED((ROWS_ACC, 16), jnp.float32),
        pltpu.SemaphoreType.DMA,
    ],
)
def _hist_pass(idx_hbm, ones_hbm, zeros_hbm, out_hbm, idx_v, ones_v, acc, sem):
    cid = lax.axis_index("c")
    sid = lax.axis_index("s")
    wid = cid * NS + sid
    pltpu.sync_copy(idx_hbm.at[wid], idx_v)
    pltpu.sync_copy(ones_hbm, ones_v)
    pltpu.sync_copy(zeros_hbm.at[pl.ds(sid * RPT_ACC, RPT_ACC)],
                    acc.at[pl.ds(sid * RPT_ACC, RPT_ACC)])
    plsc.subcore_barrier()
    pending = []
    for j in range(NCH_E):
        pending.append(
            pltpu.async_copy(ones_v, acc.at[idx_v.at[j]], sem, add=True))
        if len(pending) >= 16:
            for cp in pending:
                cp.wait()
            pending = []
    for cp in pending:
        cp.wait()
    plsc.subcore_barrier()
    pltpu.sync_copy(acc.at[pl.ds(sid * RPT_ACC, RPT_ACC)],
                    out_hbm.at[pl.ds(cid * ROWS_ACC + sid * RPT_ACC, RPT_ACC)])


@functools.partial(
    pl.kernel,
    out_type=jax.ShapeDtypeStruct((NC * ROWS_ACC, HID), jnp.float32),
    mesh=_MESH,
    compiler_params=_SC_PARAMS,
    scratch_types=[
        pltpu.VMEM((NCH_E, C), jnp.int32),
        pltpu.VMEM((NCH_E, C), jnp.int32),
        pltpu.VMEM((NBUF, C, HID), jnp.float32),
        pltpu.VMEM_SHARED((ROWS_ACC, HID), jnp.float32),
        pltpu.SemaphoreType.DMA((NBUF,)),
        pltpu.SemaphoreType.DMA((NBUF,)),
    ],
)
def _edge_pass(src_hbm, dst_hbm, y_hbm, zeros_hbm, out_hbm,
               src_v, dst_v, rows, acc, gsem, ssem):
    cid = lax.axis_index("c")
    sid = lax.axis_index("s")
    wid = cid * NS + sid
    pltpu.sync_copy(src_hbm.at[wid], src_v)
    pltpu.sync_copy(dst_hbm.at[wid], dst_v)
    pltpu.sync_copy(zeros_hbm.at[pl.ds(sid * RPT_ACC, RPT_ACC)],
                    acc.at[pl.ds(sid * RPT_ACC, RPT_ACC)])
    plsc.subcore_barrier()

    pltpu.async_copy(y_hbm.at[src_v.at[0]], rows.at[0], gsem.at[0])

    def step(i, carry):
        for k in range(NBUF):
            j = NBUF * i + k
            b1 = (k + 1) % NBUF

            @pl.when(jnp.logical_and(j >= NBUF - 1, j + 1 < NCH_E))
            def _():
                pltpu.make_async_copy(rows.at[b1], acc.at[dst_v.at[0]],
                                      ssem.at[b1]).wait()

            @pl.when(j + 1 < NCH_E)
            def _():
                pltpu.async_copy(y_hbm.at[src_v.at[j + 1]], rows.at[b1],
                                 gsem.at[b1])

            pltpu.make_async_copy(y_hbm.at[src_v.at[j]], rows.at[k],
                                  gsem.at[k]).wait()
            pltpu.async_copy(rows.at[k], acc.at[dst_v.at[j]], ssem.at[k],
                             add=True)
        return carry

    lax.fori_loop(0, NCH_E // NBUF, step, 0)
    for k in range(NBUF):
        pltpu.make_async_copy(rows.at[k], acc.at[dst_v.at[0]],
                              ssem.at[k]).wait()
    plsc.subcore_barrier()
    pltpu.sync_copy(acc.at[pl.ds(sid * RPT_ACC, RPT_ACC)],
                    out_hbm.at[pl.ds(cid * ROWS_ACC + sid * RPT_ACC, RPT_ACC)])


_BR = 1000
_GRID = N_NODES // _BR


def _dinv_of(hist_blk):
    return lax.rsqrt(hist_blk[0, :, 0:1] + hist_blk[1, :, 0:1] + 1.0)


def _mm1_body(x_ref, w_ref, o_ref):
    o_ref[...] = jnp.dot(x_ref[...], w_ref[...],
                         preferred_element_type=jnp.float32)


def _mm1(x, w1):
    return pl.pallas_call(
        _mm1_body,
        grid=(_GRID,),
        in_specs=[
            pl.BlockSpec((_BR, X_DIM), lambda i: (i, 0)),
            pl.BlockSpec((X_DIM, HID), lambda i: (0, 0)),
        ],
        out_specs=pl.BlockSpec((_BR, HID), lambda i: (i, 0)),
        out_shape=jax.ShapeDtypeStruct((N_NODES, HID), jnp.float32),
    )(x, w1)


def _scale_y_body(xw_ref, hist_ref, y_ref):
    y_ref[...] = xw_ref[...] * _dinv_of(hist_ref[...])


def _scale_y(xw, hist3):
    return pl.pallas_call(
        _scale_y_body,
        grid=(_GRID,),
        in_specs=[
            pl.BlockSpec((_BR, HID), lambda i: (i, 0)),
            pl.BlockSpec((NC, _BR, 16), lambda i: (0, i, 0)),
        ],
        out_specs=pl.BlockSpec((_BR, HID), lambda i: (i, 0)),
        out_shape=jax.ShapeDtypeStruct((N_NODES, HID), jnp.float32),
    )(xw, hist3)


def _combine_mm_body(p_ref, xw_ref, hist_ref, b_ref, w_ref, xwn_ref, yn_ref):
    dv = _dinv_of(hist_ref[...])
    raw = p_ref[0] + p_ref[1]
    h = raw * dv + xw_ref[...] * (dv * dv) + b_ref[...]
    h = jnp.maximum(h, 0.0)
    xwn = jnp.dot(h, w_ref[...], preferred_element_type=jnp.float32)
    xwn_ref[...] = xwn
    yn_ref[...] = xwn * dv


def _combine_mm(parts, xw, hist3, b, w_next):
    return pl.pallas_call(
        _combine_mm_body,
        grid=(_GRID,),
        in_specs=[
            pl.BlockSpec((NC, _BR, HID), lambda i: (0, i, 0)),
            pl.BlockSpec((_BR, HID), lambda i: (i, 0)),
            pl.BlockSpec((NC, _BR, 16), lambda i: (0, i, 0)),
            pl.BlockSpec((1, HID), lambda i: (0, 0)),
            pl.BlockSpec((HID, HID), lambda i: (0, 0)),
        ],
        out_specs=(
            pl.BlockSpec((_BR, HID), lambda i: (i, 0)),
            pl.BlockSpec((_BR, HID), lambda i: (i, 0)),
        ),
        out_shape=(
            jax.ShapeDtypeStruct((N_NODES, HID), jnp.float32),
            jax.ShapeDtypeStruct((N_NODES, HID), jnp.float32),
        ),
    )(parts, xw, hist3, b, w_next)


def _pool_final_body(p_ref, xw_ref, hist_ref, bat_ref, b_ref, lw_ref, lb_ref,
                     o_ref, acc_g):
    i = pl.program_id(0)
    dv = _dinv_of(hist_ref[...])
    raw = p_ref[0] + p_ref[1]
    h = raw * dv + xw_ref[...] * (dv * dv) + b_ref[...]
    hx = jnp.concatenate([h, jnp.ones((_BR, 1), jnp.float32)], axis=1)
    oh = (bat_ref[...] == lax.broadcasted_iota(jnp.int32, (_BR, N_GRAPHS), 1)
          ).astype(jnp.float32)
    gpart = lax.dot_general(oh, hx, (((0,), (0,)), ((), ())),
                            preferred_element_type=jnp.float32)

    @pl.when(i == 0)
    def _():
        acc_g[...] = jnp.zeros_like(acc_g)

    acc_g[...] += gpart

    a = acc_g[...]
    g = a[:, :HID] * (1.0 / jnp.maximum(a[:, HID:], 1.0))
    z = jnp.dot(g, lw_ref[...], preferred_element_type=jnp.float32)
    o_ref[...] = jax.nn.sigmoid(z + lb_ref[...])


def _pool_final(parts, xw, hist3, batch2d, b, lin_w, lin_b):
    return pl.pallas_call(
        _pool_final_body,
        grid=(_GRID,),
        in_specs=[
            pl.BlockSpec((NC, _BR, HID), lambda i: (0, i, 0)),
            pl.BlockSpec((_BR, HID), lambda i: (i, 0)),
            pl.BlockSpec((NC, _BR, 16), lambda i: (0, i, 0)),
            pl.BlockSpec((_BR, 1), lambda i: (i, 0)),
            pl.BlockSpec((1, HID), lambda i: (0, 0)),
            pl.BlockSpec((HID, 1), lambda i: (0, 0)),
            pl.BlockSpec((1, 1), lambda i: (0, 0)),
        ],
        out_specs=pl.BlockSpec((N_GRAPHS, 1), lambda i: (0, 0)),
        out_shape=jax.ShapeDtypeStruct((N_GRAPHS, 1), jnp.float32),
        scratch_shapes=[
            pltpu.VMEM((N_GRAPHS, HID + 1), jnp.float32),
        ],
    )(parts, xw, hist3, batch2d, b, lin_w, lin_b)


def kernel(x, edge_index, batch, W1, b1, W2, b2, W3, b3, lin_W, lin_b):
    n_edges = edge_index.shape[1]
    src = edge_index[0].astype(jnp.int32)
    dst = edge_index[1].astype(jnp.int32)
    batch = batch.astype(jnp.int32)

    pe = E_PAD - n_edges
    are = jnp.arange(pe, dtype=jnp.int32)
    src_p = jnp.concatenate([src, are % N_NODES]).reshape(NW, NCH_E, C)
    dst_p = jnp.concatenate([dst, N_NODES + (are % 16)]).reshape(NW, NCH_E, C)

    ones16 = jnp.ones((C, 16), jnp.float32)
    z_hist = jnp.zeros((ROWS_ACC, 16), jnp.float32)
    z_acc = jnp.zeros((ROWS_ACC, HID), jnp.float32)
    b1r = b1.reshape(1, HID)
    b2r = b2.reshape(1, HID)
    b3r = b3.reshape(1, HID)

    hist = _hist_pass(dst_p, ones16, z_hist)
    xw1 = _mm1(x, W1)
    hist3 = hist.reshape(NC, ROWS_ACC, 16)
    y1 = _scale_y(xw1, hist3)

    p1 = _edge_pass(src_p, dst_p, y1, z_acc)
    xw2, y2 = _combine_mm(p1.reshape(NC, ROWS_ACC, HID), xw1, hist3,
                          b1r, W2)
    p2 = _edge_pass(src_p, dst_p, y2, z_acc)
    xw3, y3 = _combine_mm(p2.reshape(NC, ROWS_ACC, HID), xw2, hist3,
                          b2r, W3)
    p3 = _edge_pass(src_p, dst_p, y3, z_acc)

    out = _pool_final(p3.reshape(NC, ROWS_ACC, HID), xw3, hist3,
                      batch.reshape(N_NODES, 1), b3r,
                      lin_W, lin_b.reshape(1, 1))
    return out


# speedup vs baseline: 36.0527x; 1.0376x over previous
import functools

import jax
import jax.numpy as jnp
from jax import lax
from jax.experimental import pallas as pl
from jax.experimental.pallas import tpu as pltpu
from jax.experimental.pallas import tpu_sc as plsc

N_NODES = 10000
X_DIM = 128
HID = 64
N_GRAPHS = 32

NC = 2
NS = 16
NW = NC * NS
C = 128
EPW = 10000
NCH_E = 80
IDXN = NCH_E * C
ROWS_ACC = N_NODES + 112
RPT_ACC = ROWS_ACC // NS
NBUF = 4

_MESH = plsc.VectorSubcoreMesh(core_axis_name="c", subcore_axis_name="s")
_SC_PARAMS = pltpu.CompilerParams(use_tc_tiling_on_sc=False)


def _stage_idx(edge_hbm, row, wid, idx_v, dummy_base, dummy_n):
    base = pl.multiple_of(wid * EPW, 8)
    pltpu.sync_copy(edge_hbm.at[row, pl.ds(base, EPW)],
                    idx_v.at[pl.ds(0, EPW)])
    for k in range((IDXN - EPW) // 16):
        lanes = lax.iota(jnp.int32, 16) + (16 * k)
        idx_v[pl.ds(EPW + 16 * k, 16)] = dummy_base + lanes % dummy_n


@functools.partial(
    pl.kernel,
    out_type=jax.ShapeDtypeStruct((NC * ROWS_ACC, 16), jnp.float32),
    mesh=_MESH,
    compiler_params=_SC_PARAMS,
    scratch_types=[
        pltpu.VMEM((IDXN,), jnp.int32),
        pltpu.VMEM((C, 16), jnp.float32),
        pltpu.VMEM_SHARED((ROWS_ACC, 16), jnp.float32),
        pltpu.SemaphoreType.DMA,
    ],
)
def _hist_pass(edge_hbm, ones_hbm, zeros_hbm, out_hbm, idx_v, ones_v, acc,
               sem):
    cid = lax.axis_index("c")
    sid = lax.axis_index("s")
    wid = cid * NS + sid
    _stage_idx(edge_hbm, 1, wid, idx_v, N_NODES, 112)
    pltpu.sync_copy(ones_hbm, ones_v)
    pltpu.sync_copy(zeros_hbm.at[pl.ds(sid * RPT_ACC, RPT_ACC)],
                    acc.at[pl.ds(sid * RPT_ACC, RPT_ACC)])
    plsc.subcore_barrier()
    pending = []
    for j in range(NCH_E):
        pending.append(
            pltpu.async_copy(ones_v, acc.at[idx_v.at[pl.ds(j * C, C)]], sem,
                             add=True))
        if len(pending) >= 16:
            for cp in pending:
                cp.wait()
            pending = []
    for cp in pending:
        cp.wait()
    plsc.subcore_barrier()
    pltpu.sync_copy(acc.at[pl.ds(sid * RPT_ACC, RPT_ACC)],
                    out_hbm.at[pl.ds(cid * ROWS_ACC + sid * RPT_ACC, RPT_ACC)])


@functools.partial(
    pl.kernel,
    out_type=jax.ShapeDtypeStruct((NC * ROWS_ACC, HID), jnp.float32),
    mesh=_MESH,
    compiler_params=_SC_PARAMS,
    scratch_types=[
        pltpu.VMEM((IDXN,), jnp.int32),
        pltpu.VMEM((IDXN,), jnp.int32),
        pltpu.VMEM((NBUF, C, HID), jnp.float32),
        pltpu.VMEM_SHARED((ROWS_ACC, HID), jnp.float32),
        pltpu.SemaphoreType.DMA((NBUF,)),
        pltpu.SemaphoreType.DMA((NBUF,)),
    ],
)
def _edge_pass(edge_hbm, y_hbm, zeros_hbm, out_hbm,
               src_v, dst_v, rows, acc, gsem, ssem):
    cid = lax.axis_index("c")
    sid = lax.axis_index("s")
    wid = cid * NS + sid
    _stage_idx(edge_hbm, 0, wid, src_v, 0, N_NODES)
    _stage_idx(edge_hbm, 1, wid, dst_v, N_NODES, 112)
    pltpu.sync_copy(zeros_hbm.at[pl.ds(sid * RPT_ACC, RPT_ACC)],
                    acc.at[pl.ds(sid * RPT_ACC, RPT_ACC)])
    plsc.subcore_barrier()

    def sidx(j):
        return src_v.at[pl.ds(pl.multiple_of(j * C, 8), C)]

    def didx(j):
        return dst_v.at[pl.ds(pl.multiple_of(j * C, 8), C)]

    pltpu.async_copy(y_hbm.at[sidx(0)], rows.at[0], gsem.at[0])

    def step(i, carry):
        for k in range(NBUF):
            j = NBUF * i + k
            b1 = (k + 1) % NBUF

            @pl.when(jnp.logical_and(j >= NBUF - 1, j + 1 < NCH_E))
            def _():
                pltpu.make_async_copy(rows.at[b1], acc.at[didx(0)],
                                      ssem.at[b1]).wait()

            @pl.when(j + 1 < NCH_E)
            def _():
                pltpu.async_copy(y_hbm.at[sidx(j + 1)], rows.at[b1],
                                 gsem.at[b1])

            pltpu.make_async_copy(y_hbm.at[sidx(j)], rows.at[k],
                                  gsem.at[k]).wait()
            pltpu.async_copy(rows.at[k], acc.at[didx(j)], ssem.at[k],
                             add=True)
        return carry

    lax.fori_loop(0, NCH_E // NBUF, step, 0)
    for k in range(NBUF):
        pltpu.make_async_copy(rows.at[k], acc.at[didx(0)],
                              ssem.at[k]).wait()
    plsc.subcore_barrier()
    pltpu.sync_copy(acc.at[pl.ds(sid * RPT_ACC, RPT_ACC)],
                    out_hbm.at[pl.ds(cid * ROWS_ACC + sid * RPT_ACC, RPT_ACC)])


_BR = 1000
_GRID = N_NODES // _BR


def _mm1_body(x_ref, w_ref, o_ref):
    o_ref[...] = jnp.dot(x_ref[...], w_ref[...],
                         preferred_element_type=jnp.float32)


def _mm1(x, w1):
    return pl.pallas_call(
        _mm1_body,
        grid=(_GRID,),
        in_specs=[
            pl.BlockSpec((_BR, X_DIM), lambda i: (i, 0)),
            pl.BlockSpec((X_DIM, HID), lambda i: (0, 0)),
        ],
        out_specs=pl.BlockSpec((_BR, HID), lambda i: (i, 0)),
        out_shape=jax.ShapeDtypeStruct((N_NODES, HID), jnp.float32),
    )(x, w1)


def _post_hist_body(hist_ref, xw_ref, y_ref, dv_ref):
    hp = hist_ref[...]
    dv = lax.rsqrt(hp[0, :, 0:1] + hp[1, :, 0:1] + 1.0)
    dv64 = jnp.broadcast_to(dv, (_BR, HID))
    dv_ref[...] = dv64
    y_ref[...] = xw_ref[...] * dv64


def _post_hist(hist3, xw1):
    return pl.pallas_call(
        _post_hist_body,
        grid=(_GRID,),
        in_specs=[
            pl.BlockSpec((NC, _BR, 16), lambda i: (0, i, 0)),
            pl.BlockSpec((_BR, HID), lambda i: (i, 0)),
        ],
        out_specs=(
            pl.BlockSpec((_BR, HID), lambda i: (i, 0)),
            pl.BlockSpec((_BR, HID), lambda i: (i, 0)),
        ),
        out_shape=(
            jax.ShapeDtypeStruct((N_NODES, HID), jnp.float32),
            jax.ShapeDtypeStruct((N_NODES, HID), jnp.float32),
        ),
    )(hist3, xw1)


def _combine_mm_body(p_ref, xw_ref, dv_ref, b_ref, w_ref, xwn_ref, yn_ref):
    dv = dv_ref[...]
    raw = p_ref[0] + p_ref[1]
    h = raw * dv + xw_ref[...] * (dv * dv) + b_ref[...]
    h = jnp.maximum(h, 0.0)
    xwn = jnp.dot(h, w_ref[...], preferred_element_type=jnp.float32)
    xwn_ref[...] = xwn
    yn_ref[...] = xwn * dv


def _combine_mm(parts, xw, dv64, b, w_next):
    return pl.pallas_call(
        _combine_mm_body,
        grid=(_GRID,),
        in_specs=[
            pl.BlockSpec((NC, _BR, HID), lambda i: (0, i, 0)),
            pl.BlockSpec((_BR, HID), lambda i: (i, 0)),
            pl.BlockSpec((_BR, HID), lambda i: (i, 0)),
            pl.BlockSpec((1, HID), lambda i: (0, 0)),
            pl.BlockSpec((HID, HID), lambda i: (0, 0)),
        ],
        out_specs=(
            pl.BlockSpec((_BR, HID), lambda i: (i, 0)),
            pl.BlockSpec((_BR, HID), lambda i: (i, 0)),
        ),
        out_shape=(
            jax.ShapeDtypeStruct((N_NODES, HID), jnp.float32),
            jax.ShapeDtypeStruct((N_NODES, HID), jnp.float32),
        ),
    )(parts, xw, dv64, b, w_next)


def _pool_final_body(p_ref, xw_ref, dv_ref, bat_ref, b_ref, lw_ref, lb_ref,
                     o_ref, acc_g):
    i = pl.program_id(0)
    dv = dv_ref[...]
    raw = p_ref[0] + p_ref[1]
    h = raw * dv + xw_ref[...] * (dv * dv) + b_ref[...]
    hx = jnp.concatenate([h, jnp.ones((_BR, 1), jnp.float32)], axis=1)
    oh = (bat_ref[...] == lax.broadcasted_iota(jnp.int32, (_BR, N_GRAPHS), 1)
          ).astype(jnp.float32)
    gpart = lax.dot_general(oh, hx, (((0,), (0,)), ((), ())),
                            preferred_element_type=jnp.float32)

    @pl.when(i == 0)
    def _():
        acc_g[...] = jnp.zeros_like(acc_g)

    acc_g[...] += gpart

    a = acc_g[...]
    g = a[:, :HID] * (1.0 / jnp.maximum(a[:, HID:], 1.0))
    z = jnp.dot(g, lw_ref[...], preferred_element_type=jnp.float32)
    o_ref[...] = jax.nn.sigmoid(z + lb_ref[...])


def _pool_final(parts, xw, dv64, batch2d, b, lin_w, lin_b):
    return pl.pallas_call(
        _pool_final_body,
        grid=(_GRID,),
        in_specs=[
            pl.BlockSpec((NC, _BR, HID), lambda i: (0, i, 0)),
            pl.BlockSpec((_BR, HID), lambda i: (i, 0)),
            pl.BlockSpec((_BR, HID), lambda i: (i, 0)),
            pl.BlockSpec((_BR, 1), lambda i: (i, 0)),
            pl.BlockSpec((1, HID), lambda i: (0, 0)),
            pl.BlockSpec((HID, 1), lambda i: (0, 0)),
            pl.BlockSpec((1, 1), lambda i: (0, 0)),
        ],
        out_specs=pl.BlockSpec((N_GRAPHS, 1), lambda i: (0, 0)),
        out_shape=jax.ShapeDtypeStruct((N_GRAPHS, 1), jnp.float32),
        scratch_shapes=[
            pltpu.VMEM((N_GRAPHS, HID + 1), jnp.float32),
        ],
    )(parts, xw, dv64, batch2d, b, lin_w, lin_b)


def kernel(x, edge_index, batch, W1, b1, W2, b2, W3, b3, lin_W, lin_b):
    edges = edge_index.astype(jnp.int32)
    batch = batch.astype(jnp.int32)

    ones16 = jnp.ones((C, 16), jnp.float32)
    z_hist = jnp.zeros((ROWS_ACC, 16), jnp.float32)
    z_acc = jnp.zeros((ROWS_ACC, HID), jnp.float32)
    b1r = b1.reshape(1, HID)
    b2r = b2.reshape(1, HID)
    b3r = b3.reshape(1, HID)

    hist = _hist_pass(edges, ones16, z_hist)
    xw1 = _mm1(x, W1)
    hist3 = hist.reshape(NC, ROWS_ACC, 16)
    y1, dv64 = _post_hist(hist3, xw1)

    p1 = _edge_pass(edges, y1, z_acc)
    xw2, y2 = _combine_mm(p1.reshape(NC, ROWS_ACC, HID), xw1, dv64,
                          b1r, W2)
    p2 = _edge_pass(edges, y2, z_acc)
    xw3, y3 = _combine_mm(p2.reshape(NC, ROWS_ACC, HID), xw2, dv64,
                          b2r, W3)
    p3 = _edge_pass(edges, y3, z_acc)

    out = _pool_final(p3.reshape(NC, ROWS_ACC, HID), xw3, dv64,
                      batch.reshape(N_NODES, 1), b3r,
                      lin_W, lin_b.reshape(1, 1))
    return out


# speedup vs baseline: 37.0672x; 1.0281x over previous
import functools

import jax
import jax.numpy as jnp
from jax import lax
from jax.experimental import pallas as pl
from jax.experimental.pallas import tpu as pltpu
from jax.experimental.pallas import tpu_sc as plsc

N_NODES = 10000
X_DIM = 128
HID = 64
N_GRAPHS = 32

NC = 2
NS = 16
NW = NC * NS
C = 128
EPW = 10000
NCH_E = 80
IDXN = NCH_E * C
ROWS_ACC = N_NODES + 112
RPT_ACC = ROWS_ACC // NS
NBUF = 4

_MESH = plsc.VectorSubcoreMesh(core_axis_name="c", subcore_axis_name="s")
_SC_PARAMS = pltpu.CompilerParams(use_tc_tiling_on_sc=False)


def _fill_tail(idx_v, dummy_base, dummy_n):
    for k in range((IDXN - EPW) // 16):
        lanes = lax.iota(jnp.int32, 16) + (16 * k)
        idx_v[pl.ds(EPW + 16 * k, 16)] = dummy_base + lanes % dummy_n


def _stage_idx(edge_hbm, row, wid, idx_v, dummy_base, dummy_n):
    base = pl.multiple_of(wid * EPW, 8)
    pltpu.sync_copy(edge_hbm.at[row, pl.ds(base, EPW)],
                    idx_v.at[pl.ds(0, EPW)])
    _fill_tail(idx_v, dummy_base, dummy_n)


@functools.partial(
    pl.kernel,
    out_type=jax.ShapeDtypeStruct((NC * ROWS_ACC, 16), jnp.float32),
    mesh=_MESH,
    compiler_params=_SC_PARAMS,
    scratch_types=[
        pltpu.VMEM((IDXN,), jnp.int32),
        pltpu.VMEM((C, 16), jnp.float32),
        pltpu.VMEM_SHARED((ROWS_ACC, 16), jnp.float32),
        pltpu.SemaphoreType.DMA,
    ],
)
def _hist_pass(edge_hbm, ones_hbm, zeros_hbm, out_hbm, idx_v, ones_v, acc,
               sem):
    cid = lax.axis_index("c")
    sid = lax.axis_index("s")
    wid = cid * NS + sid
    _stage_idx(edge_hbm, 1, wid, idx_v, N_NODES, 112)
    pltpu.sync_copy(ones_hbm, ones_v)
    pltpu.sync_copy(zeros_hbm.at[pl.ds(sid * RPT_ACC, RPT_ACC)],
                    acc.at[pl.ds(sid * RPT_ACC, RPT_ACC)])
    plsc.subcore_barrier()
    pending = []
    for j in range(NCH_E):
        pending.append(
            pltpu.async_copy(ones_v, acc.at[idx_v.at[pl.ds(j * C, C)]], sem,
                             add=True))
        if len(pending) >= 16:
            for cp in pending:
                cp.wait()
            pending = []
    for cp in pending:
        cp.wait()
    plsc.subcore_barrier()
    pltpu.sync_copy(acc.at[pl.ds(sid * RPT_ACC, RPT_ACC)],
                    out_hbm.at[pl.ds(cid * ROWS_ACC + sid * RPT_ACC, RPT_ACC)])


@functools.partial(
    pl.kernel,
    out_type=jax.ShapeDtypeStruct((NC * ROWS_ACC, HID), jnp.float32),
    mesh=_MESH,
    compiler_params=_SC_PARAMS,
    scratch_types=[
        pltpu.VMEM((IDXN,), jnp.int32),
        pltpu.VMEM((IDXN,), jnp.int32),
        pltpu.VMEM((NBUF, C, HID), jnp.float32),
        pltpu.VMEM_SHARED((ROWS_ACC, HID), jnp.float32),
        pltpu.SemaphoreType.DMA((NBUF,)),
        pltpu.SemaphoreType.DMA((NBUF,)),
    ],
)
def _edge_pass(edge_hbm, y_hbm, zeros_hbm, out_hbm,
               src_v, dst_v, rows, acc, gsem, ssem):
    cid = lax.axis_index("c")
    sid = lax.axis_index("s")
    wid = cid * NS + sid
    base = pl.multiple_of(wid * EPW, 8)
    cs = pltpu.async_copy(edge_hbm.at[0, pl.ds(base, EPW)],
                          src_v.at[pl.ds(0, EPW)], gsem.at[1])
    cd = pltpu.async_copy(edge_hbm.at[1, pl.ds(base, EPW)],
                          dst_v.at[pl.ds(0, EPW)], gsem.at[2])
    cz = pltpu.async_copy(zeros_hbm.at[pl.ds(sid * RPT_ACC, RPT_ACC)],
                          acc.at[pl.ds(sid * RPT_ACC, RPT_ACC)], gsem.at[3])
    _fill_tail(src_v, 0, N_NODES)
    _fill_tail(dst_v, N_NODES, 112)
    cs.wait()
    cd.wait()
    cz.wait()
    plsc.subcore_barrier()

    def sidx(j):
        return src_v.at[pl.ds(pl.multiple_of(j * C, 8), C)]

    def didx(j):
        return dst_v.at[pl.ds(pl.multiple_of(j * C, 8), C)]

    pltpu.async_copy(y_hbm.at[sidx(0)], rows.at[0], gsem.at[0])

    def step(i, carry):
        for k in range(NBUF):
            j = NBUF * i + k
            b1 = (k + 1) % NBUF

            @pl.when(jnp.logical_and(j >= NBUF - 1, j + 1 < NCH_E))
            def _():
                pltpu.make_async_copy(rows.at[b1], acc.at[didx(0)],
                                      ssem.at[b1]).wait()

            @pl.when(j + 1 < NCH_E)
            def _():
                pltpu.async_copy(y_hbm.at[sidx(j + 1)], rows.at[b1],
                                 gsem.at[b1])

            pltpu.make_async_copy(y_hbm.at[sidx(j)], rows.at[k],
                                  gsem.at[k]).wait()
            pltpu.async_copy(rows.at[k], acc.at[didx(j)], ssem.at[k],
                             add=True)
        return carry

    lax.fori_loop(0, NCH_E // NBUF, step, 0)
    for k in range(NBUF):
        pltpu.make_async_copy(rows.at[k], acc.at[didx(0)],
                              ssem.at[k]).wait()
    plsc.subcore_barrier()
    pltpu.sync_copy(acc.at[pl.ds(sid * RPT_ACC, RPT_ACC)],
                    out_hbm.at[pl.ds(cid * ROWS_ACC + sid * RPT_ACC, RPT_ACC)])


_BR = 1000
_GRID = N_NODES // _BR


def _mm1_body(x_ref, w_ref, o_ref):
    o_ref[...] = jnp.dot(x_ref[...], w_ref[...],
                         preferred_element_type=jnp.float32)


def _mm1(x, w1):
    return pl.pallas_call(
        _mm1_body,
        grid=(_GRID,),
        in_specs=[
            pl.BlockSpec((_BR, X_DIM), lambda i: (i, 0)),
            pl.BlockSpec((X_DIM, HID), lambda i: (0, 0)),
        ],
        out_specs=pl.BlockSpec((_BR, HID), lambda i: (i, 0)),
        out_shape=jax.ShapeDtypeStruct((N_NODES, HID), jnp.float32),
    )(x, w1)


def _post_hist_body(hist_ref, xw_ref, y_ref, dv_ref):
    hp = hist_ref[...]
    dv = lax.rsqrt(hp[0, :, 0:1] + hp[1, :, 0:1] + 1.0)
    dv64 = jnp.broadcast_to(dv, (_BR, HID))
    dv_ref[...] = dv64
    y_ref[...] = xw_ref[...] * dv64


def _post_hist(hist3, xw1):
    return pl.pallas_call(
        _post_hist_body,
        grid=(_GRID,),
        in_specs=[
            pl.BlockSpec((NC, _BR, 16), lambda i: (0, i, 0)),
            pl.BlockSpec((_BR, HID), lambda i: (i, 0)),
        ],
        out_specs=(
            pl.BlockSpec((_BR, HID), lambda i: (i, 0)),
            pl.BlockSpec((_BR, HID), lambda i: (i, 0)),
        ),
        out_shape=(
            jax.ShapeDtypeStruct((N_NODES, HID), jnp.float32),
            jax.ShapeDtypeStruct((N_NODES, HID), jnp.float32),
        ),
    )(hist3, xw1)


def _combine_mm_body(p_ref, y_ref, dv_ref, b_ref, w_ref, yn_ref):
    dv = dv_ref[...]
    h = (p_ref[0] + p_ref[1] + y_ref[...]) * dv + b_ref[...]
    h = jnp.maximum(h, 0.0)
    xwn = jnp.dot(h, w_ref[...], preferred_element_type=jnp.float32)
    yn_ref[...] = xwn * dv


def _combine_mm(parts, y, dv64, b, w_next):
    return pl.pallas_call(
        _combine_mm_body,
        grid=(_GRID,),
        in_specs=[
            pl.BlockSpec((NC, _BR, HID), lambda i: (0, i, 0)),
            pl.BlockSpec((_BR, HID), lambda i: (i, 0)),
            pl.BlockSpec((_BR, HID), lambda i: (i, 0)),
            pl.BlockSpec((1, HID), lambda i: (0, 0)),
            pl.BlockSpec((HID, HID), lambda i: (0, 0)),
        ],
        out_specs=pl.BlockSpec((_BR, HID), lambda i: (i, 0)),
        out_shape=jax.ShapeDtypeStruct((N_NODES, HID), jnp.float32),
    )(parts, y, dv64, b, w_next)


def _pool_final_body(p_ref, y_ref, dv_ref, bat_ref, b_ref, lw_ref, lb_ref,
                     o_ref, acc_g):
    i = pl.program_id(0)
    dv = dv_ref[...]
    h = (p_ref[0] + p_ref[1] + y_ref[...]) * dv + b_ref[...]
    hx = jnp.concatenate([h, jnp.ones((_BR, 1), jnp.float32)], axis=1)
    oh = (bat_ref[...] == lax.broadcasted_iota(jnp.int32, (_BR, N_GRAPHS), 1)
          ).astype(jnp.float32)
    gpart = lax.dot_general(oh, hx, (((0,), (0,)), ((), ())),
                            preferred_element_type=jnp.float32)

    @pl.when(i == 0)
    def _():
        acc_g[...] = jnp.zeros_like(acc_g)

    acc_g[...] += gpart

    a = acc_g[...]
    g = a[:, :HID] * (1.0 / jnp.maximum(a[:, HID:], 1.0))
    z = jnp.dot(g, lw_ref[...], preferred_element_type=jnp.float32)
    o_ref[...] = jax.nn.sigmoid(z + lb_ref[...])


def _pool_final(parts, y, dv64, batch2d, b, lin_w, lin_b):
    return pl.pallas_call(
        _pool_final_body,
        grid=(_GRID,),
        in_specs=[
            pl.BlockSpec((NC, _BR, HID), lambda i: (0, i, 0)),
            pl.BlockSpec((_BR, HID), lambda i: (i, 0)),
            pl.BlockSpec((_BR, HID), lambda i: (i, 0)),
            pl.BlockSpec((_BR, 1), lambda i: (i, 0)),
            pl.BlockSpec((1, HID), lambda i: (0, 0)),
            pl.BlockSpec((HID, 1), lambda i: (0, 0)),
            pl.BlockSpec((1, 1), lambda i: (0, 0)),
        ],
        out_specs=pl.BlockSpec((N_GRAPHS, 1), lambda i: (0, 0)),
        out_shape=jax.ShapeDtypeStruct((N_GRAPHS, 1), jnp.float32),
        scratch_shapes=[
            pltpu.VMEM((N_GRAPHS, HID + 1), jnp.float32),
        ],
    )(parts, y, dv64, batch2d, b, lin_w, lin_b)


def kernel(x, edge_index, batch, W1, b1, W2, b2, W3, b3, lin_W, lin_b):
    edges = edge_index.astype(jnp.int32)
    batch = batch.astype(jnp.int32)

    ones16 = jnp.ones((C, 16), jnp.float32)
    z_hist = jnp.zeros((ROWS_ACC, 16), jnp.float32)
    z_acc = jnp.zeros((ROWS_ACC, HID), jnp.float32)
    b1r = b1.reshape(1, HID)
    b2r = b2.reshape(1, HID)
    b3r = b3.reshape(1, HID)

    hist = _hist_pass(edges, ones16, z_hist)
    xw1 = _mm1(x, W1)
    hist3 = hist.reshape(NC, ROWS_ACC, 16)
    y1, dv64 = _post_hist(hist3, xw1)

    p1 = _edge_pass(edges, y1, z_acc)
    y2 = _combine_mm(p1.reshape(NC, ROWS_ACC, HID), y1, dv64,
                     b1r, W2)
    p2 = _edge_pass(edges, y2, z_acc)
    y3 = _combine_mm(p2.reshape(NC, ROWS_ACC, HID), y2, dv64,
                     b2r, W3)
    p3 = _edge_pass(edges, y3, z_acc)

    out = _pool_final(p3.reshape(NC, ROWS_ACC, HID), y3, dv64,
                      batch.reshape(N_NODES, 1), b3r,
                      lin_W, lin_b.reshape(1, 1))
    return out
